# baseline (speedup 1.0000x reference)
import functools

import jax
import jax.numpy as jnp
from jax import lax
from jax.experimental import pallas as pl
from jax.experimental.pallas import tpu as pltpu
from jax.experimental.pallas import tpu_sc as plsc

N = 10000
E = 160000
EP = E // 2
DIN = 128
HID = 64
HEADS = 4
EDIM = 16
DOUT = 128
NL = 4
HC = HEADS * HID
CW = HID + 16
CSZ = 128
NCHUNK = EP // CSZ
NSUB = 16
BN_EPS = 1e-5
SM_EPS = 1e-16
ISQ = 0.125


BLK = 2000


def _accum_stats(i, y, s_ref, q_ref):
    s = jnp.sum(y, axis=0, keepdims=True)
    q = jnp.sum(y * y, axis=0, keepdims=True)

    @pl.when(i == 0)
    def _():
        s_ref[...] = s
        q_ref[...] = q

    @pl.when(i > 0)
    def _():
        s_ref[...] = s_ref[...] + s
        q_ref[...] = q_ref[...] + q


def _bn_from_stats(y, s, q, g, be, n):
    m = s / n
    v = q / n - m * m
    return jnp.maximum(g * (y - m) * lax.rsqrt(v + BN_EPS) + be, 0.0)


def _mm_stats_body(x_ref, w, b, y_ref, s_ref, q_ref):
    y = jnp.dot(x_ref[...], w[...], preferred_element_type=jnp.float32) + b[...]
    y_ref[...] = y
    _accum_stats(pl.program_id(0), y, s_ref, q_ref)


def _mm_stats(x, w, b):
    n, di = x.shape
    do = w.shape[1]
    return pl.pallas_call(
        _mm_stats_body,
        grid=(n // BLK,),
        in_specs=[pl.BlockSpec((BLK, di), lambda i: (i, 0)),
                  pl.BlockSpec((di, do), lambda i: (0, 0)),
                  pl.BlockSpec((1, do), lambda i: (0, 0))],
        out_specs=[pl.BlockSpec((BLK, do), lambda i: (i, 0)),
                   pl.BlockSpec((1, do), lambda i: (0, 0)),
                   pl.BlockSpec((1, do), lambda i: (0, 0))],
        out_shape=[jax.ShapeDtypeStruct((n, do), jnp.float32),
                   jax.ShapeDtypeStruct((1, do), jnp.float32),
                   jax.ShapeDtypeStruct((1, do), jnp.float32)],
    )(x, w, b)


def _bn_mm_body(y_ref, s_ref, q_ref, g, be, w, b, y2_ref, s2_ref, q2_ref, *, n):
    z = _bn_from_stats(y_ref[...], s_ref[...], q_ref[...], g[...], be[...], n)
    y2 = jnp.dot(z, w[...], preferred_element_type=jnp.float32) + b[...]
    y2_ref[...] = y2
    _accum_stats(pl.program_id(0), y2, s2_ref, q2_ref)


def _bn_mm(y, s, q, g, be, w, b):
    n, di = y.shape
    do = w.shape[1]
    return pl.pallas_call(
        functools.partial(_bn_mm_body, n=float(n)),
        grid=(n // BLK,),
        in_specs=[pl.BlockSpec((BLK, di), lambda i: (i, 0)),
                  pl.BlockSpec((1, di), lambda i: (0, 0)),
                  pl.BlockSpec((1, di), lambda i: (0, 0)),
                  pl.BlockSpec((1, di), lambda i: (0, 0)),
                  pl.BlockSpec((1, di), lambda i: (0, 0)),
                  pl.BlockSpec((di, do), lambda i: (0, 0)),
                  pl.BlockSpec((1, do), lambda i: (0, 0))],
        out_specs=[pl.BlockSpec((BLK, do), lambda i: (i, 0)),
                   pl.BlockSpec((1, do), lambda i: (0, 0)),
                   pl.BlockSpec((1, do), lambda i: (0, 0))],
        out_shape=[jax.ShapeDtypeStruct((n, do), jnp.float32),
                   jax.ShapeDtypeStruct((1, do), jnp.float32),
                   jax.ShapeDtypeStruct((1, do), jnp.float32)],
    )(y, s, q, g, be, w, b)


def _bn_apply_body(y_ref, s_ref, q_ref, g, be, o_ref, *, n):
    o_ref[...] = _bn_from_stats(y_ref[...], s_ref[...], q_ref[...],
                                g[...], be[...], n)


def _bn_apply(y, s, q, g, be):
    n, do = y.shape
    return pl.pallas_call(
        functools.partial(_bn_apply_body, n=float(n)),
        grid=(n // BLK,),
        in_specs=[pl.BlockSpec((BLK, do), lambda i: (i, 0)),
                  pl.BlockSpec((1, do), lambda i: (0, 0)),
                  pl.BlockSpec((1, do), lambda i: (0, 0)),
                  pl.BlockSpec((1, do), lambda i: (0, 0)),
                  pl.BlockSpec((1, do), lambda i: (0, 0))],
        out_specs=pl.BlockSpec((BLK, do), lambda i: (i, 0)),
        out_shape=jax.ShapeDtypeStruct((n, do), jnp.float32),
    )(y, s, q, g, be)


def _fnn(x, p, dout):
    v2 = lambda t: t.reshape(1, -1)
    y, s, q = _mm_stats(x, p['W'][0], v2(p['b'][0]))
    y, s, q = _bn_mm(y, s, q, v2(p['g'][0]), v2(p['be'][0]),
                     p['W'][1], v2(p['b'][1]))
    y, s, q = _bn_mm(y, s, q, v2(p['g'][1]), v2(p['be'][1]),
                     p['W'][2], v2(p['b'][2]))
    return _bn_apply(y, s, q, v2(p['g'][2]), v2(p['be'][2]))


def _qkv_body(h_ref, wq, bq, wk, bk, wv, bv, oq, okv):
    h = h_ref[...]
    yq = jnp.dot(h, wq[...], preferred_element_type=jnp.float32) + bq[...]
    yk = jnp.dot(h, wk[...], preferred_element_type=jnp.float32) + bk[...]
    yv = jnp.dot(h, wv[...], preferred_element_type=jnp.float32) + bv[...]
    for hh in range(HEADS):
        sl = slice(hh * HID, (hh + 1) * HID)
        oq[hh, :, :] = yq[:, sl]
        okv[hh, :, :HID] = yk[:, sl]
        okv[hh, :, HID:] = yv[:, sl]


def _qkv(h, p):
    blk = 2000
    wspec = pl.BlockSpec((HC, HC), lambda i: (0, 0))
    bspec = pl.BlockSpec((1, HC), lambda i: (0, 0))
    return pl.pallas_call(
        _qkv_body,
        grid=(N // blk,),
        in_specs=[pl.BlockSpec((blk, HC), lambda i: (i, 0)),
                  wspec, bspec, wspec, bspec, wspec, bspec],
        out_specs=[pl.BlockSpec((HEADS, blk, HID), lambda i: (0, i, 0)),
                   pl.BlockSpec((HEADS, blk, 2 * HID), lambda i: (0, i, 0))],
        out_shape=[jax.ShapeDtypeStruct((HEADS, N, HID), jnp.float32),
                   jax.ShapeDtypeStruct((HEADS, N, 2 * HID), jnp.float32)],
    )(h, p['Wq'], p['bq'].reshape(1, -1), p['Wk'], p['bk'].reshape(1, -1),
      p['Wv'], p['bv'].reshape(1, -1))


def _eproj_body(ea_ref, we, o_ref):
    y = jnp.dot(ea_ref[...], we[...], preferred_element_type=jnp.float32)
    for hh in range(HEADS):
        o_ref[hh, :, :] = y[:, hh * HID:(hh + 1) * HID].T


def _eproj(ea, we):
    blk = 3200
    return pl.pallas_call(
        _eproj_body,
        grid=(EP // blk,),
        in_specs=[pl.BlockSpec((blk, EDIM), lambda i: (i, 0)),
                  pl.BlockSpec((EDIM, HC), lambda i: (0, 0))],
        out_specs=pl.BlockSpec((HEADS, HID, blk), lambda i: (0, 0, i)),
        out_shape=jax.ShapeDtypeStruct((HEADS, HID, EP), jnp.float32),
    )(ea, we)


def _post_mm_body(acc0_ref, acc1_ref, h_ref, ws, bs, y_ref, s_ref, q_ref):
    pieces = []
    for acc_ref in (acc0_ref, acc1_ref):
        acc = acc_ref[...]
        for c in range(2):
            num = acc[c, :, :HID]
            den = acc[c, :, HID:HID + 1] + SM_EPS
            pieces.append(num / den)
    att = jnp.concatenate(pieces, axis=1)
    y = att + jnp.dot(h_ref[...], ws[...],
                      preferred_element_type=jnp.float32) + bs[...]
    y_ref[...] = y
    _accum_stats(pl.program_id(0), y, s_ref, q_ref)


def _post(acc0, acc1, h, p):
    aspec = pl.BlockSpec((2, BLK, CW), lambda i: (0, i, 0))
    y, s, q = pl.pallas_call(
        _post_mm_body,
        grid=(N // BLK,),
        in_specs=[aspec, aspec,
                  pl.BlockSpec((BLK, HC), lambda i: (i, 0)),
                  pl.BlockSpec((HC, HC), lambda i: (0, 0)),
                  pl.BlockSpec((1, HC), lambda i: (0, 0))],
        out_specs=[pl.BlockSpec((BLK, HC), lambda i: (i, 0)),
                   pl.BlockSpec((1, HC), lambda i: (0, 0)),
                   pl.BlockSpec((1, HC), lambda i: (0, 0))],
        out_shape=[jax.ShapeDtypeStruct((N, HC), jnp.float32),
                   jax.ShapeDtypeStruct((1, HC), jnp.float32),
                   jax.ShapeDtypeStruct((1, HC), jnp.float32)],
    )(acc0, acc1, h, p['Ws'], p['bs'].reshape(1, -1))
    return _bn_apply(y, s, q, p['bng'].reshape(1, -1), p['bnb'].reshape(1, -1))


def _sc_attn(q4, kv4, e4t, src, dst, p):
    mesh = plsc.VectorSubcoreMesh(core_axis_name="c", subcore_axis_name="s")
    cp = pltpu.CompilerParams(needs_layout_passes=False,
                              use_tc_tiling_on_sc=False,
                              disable_bounds_checks=True)

    @functools.partial(
        pl.kernel,
        out_type=jax.ShapeDtypeStruct((2, N, CW), jnp.float32),
        mesh=mesh,
        compiler_params=cp,
        scratch_types=[
            [pltpu.VMEM((CSZ,), jnp.int32)] * 2,
            [pltpu.VMEM((CSZ,), jnp.int32)] * 2,
            [pltpu.VMEM((CSZ,), jnp.int32)] * 2,
            [pltpu.VMEM((CSZ,), jnp.int32)] * 2,
            [pltpu.VMEM((CSZ, HID), jnp.float32)] * 2,
            [pltpu.VMEM((CSZ, 2 * HID), jnp.float32)] * 2,
            [pltpu.VMEM((HID, CSZ), jnp.float32)] * 2,
            pltpu.VMEM((CSZ, CW), jnp.float32),
            pltpu.VMEM_SHARED((N, CW), jnp.float32),
            [pltpu.SemaphoreType.DMA] * 2,
        ],
    )
    def k(q_hbm, kv_hbm, e_hbm, s_hbm, d_hbm, o_hbm,
          svb, dvb, sab, dab, qb, kvb, ebt, cb, table, sem):
        cid = lax.axis_index("c")
        sid = lax.axis_index("s")
        zero16 = jnp.zeros((16,), jnp.float32)

        @pl.loop(0, CSZ)
        def _(r):
            for j in range(CW // 16):
                cb[r, pl.ds(16 * j, 16)] = zero16

        row0 = sid * 624
        for i in range(5):
            pltpu.sync_copy(cb, table.at[pl.ds(row0 + i * 128, 128)])
        plsc.subcore_barrier()

        ioff = lax.iota(jnp.int32, 16)
        head = 2 * p + cid
        coff = head * N
        nchunks = (NCHUNK + NSUB - 1 - sid) // NSUB

        def issue_load(ci, b):
            base = (sid + ci * NSUB) * CSZ
            pltpu.sync_copy(s_hbm.at[pl.ds(base, CSZ)], svb[b])
            pltpu.sync_copy(d_hbm.at[pl.ds(base, CSZ)], dvb[b])
            for j in range(CSZ // 16):
                sl = pl.ds(16 * j, 16)
                sab[b][sl] = svb[b][sl] + coff
                dab[b][sl] = dvb[b][sl] + coff
            pltpu.async_copy(kv_hbm.at[sab[b]], kvb[b], sem[b])
            pltpu.async_copy(q_hbm.at[dab[b]], qb[b], sem[b])
            pltpu.async_copy(e_hbm.at[head, :, pl.ds(base, CSZ)], ebt[b],
                             sem[b])

        def wait_load(ci, b):
            base = (sid + ci * NSUB) * CSZ
            pltpu.make_async_copy(kv_hbm.at[sab[b]], kvb[b], sem[b]).wait()
            pltpu.make_async_copy(q_hbm.at[dab[b]], qb[b], sem[b]).wait()
            pltpu.make_async_copy(e_hbm.at[head, :, pl.ds(base, CSZ)],
                                  ebt[b], sem[b]).wait()

        def compute_chunk(b):
            kvb_, qb_, ebt_ = kvb[b], qb[b], ebt[b]

            def group_body(g, carry2):
                g16 = g * 16
                e16 = g16 + ioff
                acc0 = jnp.zeros((16,), jnp.float32)
                acc1 = jnp.zeros((16,), jnp.float32)
                for f in range(0, HID, 2):
                    c0 = jnp.full((16,), f, jnp.int32)
                    c1 = jnp.full((16,), f + 1, jnp.int32)
                    ge0 = ebt_[f, pl.ds(g16, 16)]
                    gq0 = plsc.load_gather(qb_, [e16, c0])
                    gk0 = plsc.load_gather(kvb_, [e16, c0])
                    acc0 = acc0 + gq0 * (gk0 + ge0)
                    ge1 = ebt_[f + 1, pl.ds(g16, 16)]
                    gq1 = plsc.load_gather(qb_, [e16, c1])
                    gk1 = plsc.load_gather(kvb_, [e16, c1])
                    acc1 = acc1 + gq1 * (gk1 + ge1)
                w = jnp.exp((acc0 + acc1) * ISQ)
                for f in range(HID):
                    cf = jnp.full((16,), f, jnp.int32)
                    ge = ebt_[f, pl.ds(g16, 16)]
                    gv = plsc.load_gather(kvb_, [e16, jnp.full((16,), HID + f, jnp.int32)])
                    plsc.store_scatter(cb, [e16, cf], (gv + ge) * w)
                plsc.store_scatter(cb, [e16, jnp.full((16,), HID, jnp.int32)], w)
                return carry2

            lax.fori_loop(0, CSZ // 16, group_body, 0)
            pltpu.sync_copy(cb, table.at[dvb[b]], add=True)

        issue_load(0, 0)

        @pl.when(1 < nchunks)
        def _():
            issue_load(1, 1)

        def pipe_body(it, carry):
            for b in range(2):
                ci = 2 * it + b

                @pl.when(ci < nchunks)
                def _():
                    wait_load(ci, b)
                    compute_chunk(b)

                    @pl.when(ci + 2 < nchunks)
                    def _():
                        issue_load(ci + 2, b)
            return carry

        lax.fori_loop(0, (nchunks + 1) // 2, pipe_body, 0)
        plsc.subcore_barrier()
        for i in range(5):
            sl = pl.ds(row0 + i * 128, 128)
            pltpu.sync_copy(table.at[sl], o_hbm.at[cid, sl])

    return k(q4, kv4, e4t, src, dst)


def kernel(x, edge_attr, params, edge_index):
    convs = params['convs']
    src_e = edge_index[0, 0::2]
    dst_e = edge_index[1, 0::2]
    src_o = edge_index[0, 1::2]
    dst_o = edge_index[1, 1::2]
    ea_e = edge_attr[0::2]
    ea_o = edge_attr[1::2]

    eprojs = [
        _eproj(ea_e if l % 2 == 0 else ea_o, convs[l]['We'])
        for l in range(NL)
    ]

    h = _fnn(x, params['fnn_in'], HC)
    for l in range(NL):
        p = convs[l]
        s, d = (src_e, dst_e) if l % 2 == 0 else (src_o, dst_o)
        q4, kv4 = _qkv(h, p)
        q4 = q4.reshape(HEADS * N, HID)
        kv4 = kv4.reshape(HEADS * N, 2 * HID)
        acc0 = _sc_attn(q4, kv4, eprojs[l], s, d, 0)
        acc1 = _sc_attn(q4, kv4, eprojs[l], s, d, 1)
        h = _post(acc0, acc1, h, p)
    return _fnn(h, params['fnn_out'], DOUT)


# speedup vs baseline: 3.4980x; 3.4980x over previous
import functools

import jax
import jax.numpy as jnp
from jax import lax
from jax.experimental import pallas as pl
from jax.experimental.pallas import tpu as pltpu
from jax.experimental.pallas import tpu_sc as plsc

N = 10000
E = 160000
EP = E // 2
DIN = 128
HID = 64
HEADS = 4
EDIM = 16
DOUT = 128
NL = 4
HC = HEADS * HID
CW = HID + 16
CSZ = 128
NCHUNK = EP // CSZ
NSUB = 16
BN_EPS = 1e-5
SM_EPS = 1e-16
ISQ = 0.125


BLK = 2000


def _accum_stats(i, y, s_ref, q_ref):
    s = jnp.sum(y, axis=0, keepdims=True)
    q = jnp.sum(y * y, axis=0, keepdims=True)

    @pl.when(i == 0)
    def _():
        s_ref[...] = s
        q_ref[...] = q

    @pl.when(i > 0)
    def _():
        s_ref[...] = s_ref[...] + s
        q_ref[...] = q_ref[...] + q


def _bn_from_stats(y, s, q, g, be, n):
    m = s / n
    v = q / n - m * m
    return jnp.maximum(g * (y - m) * lax.rsqrt(v + BN_EPS) + be, 0.0)


def _mm_stats_body(x_ref, w, b, y_ref, s_ref, q_ref):
    y = jnp.dot(x_ref[...], w[...], preferred_element_type=jnp.float32) + b[...]
    y_ref[...] = y
    _accum_stats(pl.program_id(0), y, s_ref, q_ref)


def _mm_stats(x, w, b):
    n, di = x.shape
    do = w.shape[1]
    return pl.pallas_call(
        _mm_stats_body,
        grid=(n // BLK,),
        in_specs=[pl.BlockSpec((BLK, di), lambda i: (i, 0)),
                  pl.BlockSpec((di, do), lambda i: (0, 0)),
                  pl.BlockSpec((1, do), lambda i: (0, 0))],
        out_specs=[pl.BlockSpec((BLK, do), lambda i: (i, 0)),
                   pl.BlockSpec((1, do), lambda i: (0, 0)),
                   pl.BlockSpec((1, do), lambda i: (0, 0))],
        out_shape=[jax.ShapeDtypeStruct((n, do), jnp.float32),
                   jax.ShapeDtypeStruct((1, do), jnp.float32),
                   jax.ShapeDtypeStruct((1, do), jnp.float32)],
    )(x, w, b)


def _bn_mm_body(y_ref, s_ref, q_ref, g, be, w, b, y2_ref, s2_ref, q2_ref, *, n):
    z = _bn_from_stats(y_ref[...], s_ref[...], q_ref[...], g[...], be[...], n)
    y2 = jnp.dot(z, w[...], preferred_element_type=jnp.float32) + b[...]
    y2_ref[...] = y2
    _accum_stats(pl.program_id(0), y2, s2_ref, q2_ref)


def _bn_mm(y, s, q, g, be, w, b):
    n, di = y.shape
    do = w.shape[1]
    return pl.pallas_call(
        functools.partial(_bn_mm_body, n=float(n)),
        grid=(n // BLK,),
        in_specs=[pl.BlockSpec((BLK, di), lambda i: (i, 0)),
                  pl.BlockSpec((1, di), lambda i: (0, 0)),
                  pl.BlockSpec((1, di), lambda i: (0, 0)),
                  pl.BlockSpec((1, di), lambda i: (0, 0)),
                  pl.BlockSpec((1, di), lambda i: (0, 0)),
                  pl.BlockSpec((di, do), lambda i: (0, 0)),
                  pl.BlockSpec((1, do), lambda i: (0, 0))],
        out_specs=[pl.BlockSpec((BLK, do), lambda i: (i, 0)),
                   pl.BlockSpec((1, do), lambda i: (0, 0)),
                   pl.BlockSpec((1, do), lambda i: (0, 0))],
        out_shape=[jax.ShapeDtypeStruct((n, do), jnp.float32),
                   jax.ShapeDtypeStruct((1, do), jnp.float32),
                   jax.ShapeDtypeStruct((1, do), jnp.float32)],
    )(y, s, q, g, be, w, b)


def _bn_apply_body(y_ref, s_ref, q_ref, g, be, o_ref, *, n):
    o_ref[...] = _bn_from_stats(y_ref[...], s_ref[...], q_ref[...],
                                g[...], be[...], n)


def _bn_apply(y, s, q, g, be):
    n, do = y.shape
    return pl.pallas_call(
        functools.partial(_bn_apply_body, n=float(n)),
        grid=(n // BLK,),
        in_specs=[pl.BlockSpec((BLK, do), lambda i: (i, 0)),
                  pl.BlockSpec((1, do), lambda i: (0, 0)),
                  pl.BlockSpec((1, do), lambda i: (0, 0)),
                  pl.BlockSpec((1, do), lambda i: (0, 0)),
                  pl.BlockSpec((1, do), lambda i: (0, 0))],
        out_specs=pl.BlockSpec((BLK, do), lambda i: (i, 0)),
        out_shape=jax.ShapeDtypeStruct((n, do), jnp.float32),
    )(y, s, q, g, be)


def _fnn(x, p, dout):
    v2 = lambda t: t.reshape(1, -1)
    y, s, q = _mm_stats(x, p['W'][0], v2(p['b'][0]))
    y, s, q = _bn_mm(y, s, q, v2(p['g'][0]), v2(p['be'][0]),
                     p['W'][1], v2(p['b'][1]))
    y, s, q = _bn_mm(y, s, q, v2(p['g'][1]), v2(p['be'][1]),
                     p['W'][2], v2(p['b'][2]))
    return _bn_apply(y, s, q, v2(p['g'][2]), v2(p['be'][2]))


def _qkv_body(h_ref, wq, bq, wk, bk, wv, bv, oq, okv):
    h = h_ref[...]
    yq = jnp.dot(h, wq[...], preferred_element_type=jnp.float32) + bq[...]
    yk = jnp.dot(h, wk[...], preferred_element_type=jnp.float32) + bk[...]
    yv = jnp.dot(h, wv[...], preferred_element_type=jnp.float32) + bv[...]
    for hh in range(HEADS):
        sl = slice(hh * HID, (hh + 1) * HID)
        oq[hh, :, :] = yq[:, sl]
        okv[hh, :, :HID] = yk[:, sl]
        okv[hh, :, HID:] = yv[:, sl]


def _qkv(h, p):
    blk = 2000
    wspec = pl.BlockSpec((HC, HC), lambda i: (0, 0))
    bspec = pl.BlockSpec((1, HC), lambda i: (0, 0))
    return pl.pallas_call(
        _qkv_body,
        grid=(N // blk,),
        in_specs=[pl.BlockSpec((blk, HC), lambda i: (i, 0)),
                  wspec, bspec, wspec, bspec, wspec, bspec],
        out_specs=[pl.BlockSpec((HEADS, blk, HID), lambda i: (0, i, 0)),
                   pl.BlockSpec((HEADS, blk, 2 * HID), lambda i: (0, i, 0))],
        out_shape=[jax.ShapeDtypeStruct((HEADS, N, HID), jnp.float32),
                   jax.ShapeDtypeStruct((HEADS, N, 2 * HID), jnp.float32)],
    )(h, p['Wq'], p['bq'].reshape(1, -1), p['Wk'], p['bk'].reshape(1, -1),
      p['Wv'], p['bv'].reshape(1, -1))


def _eproj_body(ea_ref, we, o_ref):
    y = jnp.dot(ea_ref[...], we[...], preferred_element_type=jnp.float32)
    for hh in range(HEADS):
        o_ref[hh, :, :] = y[:, hh * HID:(hh + 1) * HID].T


def _eproj(ea, we):
    blk = 3200
    return pl.pallas_call(
        _eproj_body,
        grid=(EP // blk,),
        in_specs=[pl.BlockSpec((blk, EDIM), lambda i: (i, 0)),
                  pl.BlockSpec((EDIM, HC), lambda i: (0, 0))],
        out_specs=pl.BlockSpec((HEADS, HID, blk), lambda i: (0, 0, i)),
        out_shape=jax.ShapeDtypeStruct((HEADS, HID, EP), jnp.float32),
    )(ea, we)


def _post_mm_body(acc0_ref, acc1_ref, h_ref, ws, bs, y_ref, s_ref, q_ref):
    pieces = []
    for acc_ref in (acc0_ref, acc1_ref):
        acc = acc_ref[...]
        for c in range(2):
            num = acc[c, :, :HID]
            den = acc[c, :, HID:HID + 1] + SM_EPS
            pieces.append(num / den)
    att = jnp.concatenate(pieces, axis=1)
    y = att + jnp.dot(h_ref[...], ws[...],
                      preferred_element_type=jnp.float32) + bs[...]
    y_ref[...] = y
    _accum_stats(pl.program_id(0), y, s_ref, q_ref)


def _post(acc0, acc1, h, p):
    aspec = pl.BlockSpec((2, BLK, CW), lambda i: (0, i, 0))
    y, s, q = pl.pallas_call(
        _post_mm_body,
        grid=(N // BLK,),
        in_specs=[aspec, aspec,
                  pl.BlockSpec((BLK, HC), lambda i: (i, 0)),
                  pl.BlockSpec((HC, HC), lambda i: (0, 0)),
                  pl.BlockSpec((1, HC), lambda i: (0, 0))],
        out_specs=[pl.BlockSpec((BLK, HC), lambda i: (i, 0)),
                   pl.BlockSpec((1, HC), lambda i: (0, 0)),
                   pl.BlockSpec((1, HC), lambda i: (0, 0))],
        out_shape=[jax.ShapeDtypeStruct((N, HC), jnp.float32),
                   jax.ShapeDtypeStruct((1, HC), jnp.float32),
                   jax.ShapeDtypeStruct((1, HC), jnp.float32)],
    )(acc0, acc1, h, p['Ws'], p['bs'].reshape(1, -1))
    return _bn_apply(y, s, q, p['bng'].reshape(1, -1), p['bnb'].reshape(1, -1))


def _sc_attn(q4, kv4, e4t, src, dst, p):
    mesh = plsc.VectorSubcoreMesh(core_axis_name="c", subcore_axis_name="s")
    cp = pltpu.CompilerParams(needs_layout_passes=False,
                              use_tc_tiling_on_sc=False,
                              disable_bounds_checks=True)

    @functools.partial(
        pl.kernel,
        out_type=jax.ShapeDtypeStruct((2, N, CW), jnp.float32),
        mesh=mesh,
        compiler_params=cp,
        scratch_types=[
            [pltpu.VMEM((CSZ,), jnp.int32)] * 2,
            [pltpu.VMEM((CSZ,), jnp.int32)] * 2,
            [pltpu.VMEM((CSZ,), jnp.int32)] * 2,
            [pltpu.VMEM((CSZ,), jnp.int32)] * 2,
            [pltpu.VMEM((CSZ, HID), jnp.float32)] * 2,
            [pltpu.VMEM((CSZ, 2 * HID), jnp.float32)] * 2,
            [pltpu.VMEM((HID, CSZ), jnp.float32)] * 2,
            pltpu.VMEM((CSZ, CW), jnp.float32),
            pltpu.VMEM_SHARED((N, CW), jnp.float32),
            [pltpu.SemaphoreType.DMA] * 2,
        ],
    )
    def k(q_hbm, kv_hbm, e_hbm, s_hbm, d_hbm, o_hbm,
          svb, dvb, sab, dab, qb, kvb, ebt, cb, table, sem):
        cid = lax.axis_index("c")
        sid = lax.axis_index("s")
        zero16 = jnp.zeros((16,), jnp.float32)

        @pl.loop(0, CSZ)
        def _(r):
            for j in range(CW // 16):
                cb[r, pl.ds(16 * j, 16)] = zero16

        row0 = sid * 624
        for i in range(5):
            pltpu.sync_copy(cb, table.at[pl.ds(row0 + i * 128, 128)])
        plsc.subcore_barrier()

        ioff = lax.iota(jnp.int32, 16)
        head = 2 * p + cid
        coff = head * N
        nchunks = (NCHUNK + NSUB - 1 - sid) // NSUB

        def issue_load(ci, b):
            base = (sid + ci * NSUB) * CSZ
            pltpu.sync_copy(s_hbm.at[pl.ds(base, CSZ)], svb[b])
            pltpu.sync_copy(d_hbm.at[pl.ds(base, CSZ)], dvb[b])
            for j in range(CSZ // 16):
                sl = pl.ds(16 * j, 16)
                sab[b][sl] = svb[b][sl] + coff
                dab[b][sl] = dvb[b][sl] + coff
            pltpu.async_copy(kv_hbm.at[sab[b]], kvb[b], sem[b])
            pltpu.async_copy(q_hbm.at[dab[b]], qb[b], sem[b])
            pltpu.async_copy(e_hbm.at[head, :, pl.ds(base, CSZ)], ebt[b],
                             sem[b])

        def wait_load(ci, b):
            base = (sid + ci * NSUB) * CSZ
            pltpu.make_async_copy(kv_hbm.at[sab[b]], kvb[b], sem[b]).wait()
            pltpu.make_async_copy(q_hbm.at[dab[b]], qb[b], sem[b]).wait()
            pltpu.make_async_copy(e_hbm.at[head, :, pl.ds(base, CSZ)],
                                  ebt[b], sem[b]).wait()

        def compute_chunk(b):
            kvb_, qb_, ebt_ = kvb[b], qb[b], ebt[b]

            @functools.partial(plsc.parallel_loop, 0, CSZ // 16)
            def _(g):
                g16 = g * 16
                e16 = g16 + ioff
                acc0 = jnp.zeros((16,), jnp.float32)
                acc1 = jnp.zeros((16,), jnp.float32)
                for f in range(0, HID, 2):
                    c0 = jnp.full((16,), f, jnp.int32)
                    c1 = jnp.full((16,), f + 1, jnp.int32)
                    ge0 = ebt_[f, pl.ds(g16, 16)]
                    gq0 = plsc.load_gather(qb_, [e16, c0])
                    gk0 = plsc.load_gather(kvb_, [e16, c0])
                    acc0 = acc0 + gq0 * (gk0 + ge0)
                    ge1 = ebt_[f + 1, pl.ds(g16, 16)]
                    gq1 = plsc.load_gather(qb_, [e16, c1])
                    gk1 = plsc.load_gather(kvb_, [e16, c1])
                    acc1 = acc1 + gq1 * (gk1 + ge1)
                w = jnp.exp((acc0 + acc1) * ISQ)
                for f in range(HID):
                    cf = jnp.full((16,), f, jnp.int32)
                    ge = ebt_[f, pl.ds(g16, 16)]
                    gv = plsc.load_gather(kvb_, [e16, jnp.full((16,), HID + f, jnp.int32)])
                    plsc.store_scatter(cb, [e16, cf], (gv + ge) * w)
                plsc.store_scatter(cb, [e16, jnp.full((16,), HID, jnp.int32)], w)
            pltpu.sync_copy(cb, table.at[dvb[b]], add=True)

        issue_load(0, 0)

        @pl.when(1 < nchunks)
        def _():
            issue_load(1, 1)

        def pipe_body(it, carry):
            for b in range(2):
                ci = 2 * it + b

                @pl.when(ci < nchunks)
                def _():
                    wait_load(ci, b)
                    compute_chunk(b)

                    @pl.when(ci + 2 < nchunks)
                    def _():
                        issue_load(ci + 2, b)
            return carry

        lax.fori_loop(0, (nchunks + 1) // 2, pipe_body, 0)
        plsc.subcore_barrier()
        for i in range(5):
            sl = pl.ds(row0 + i * 128, 128)
            pltpu.sync_copy(table.at[sl], o_hbm.at[cid, sl])

    return k(q4, kv4, e4t, src, dst)


def kernel(x, edge_attr, params, edge_index):
    convs = params['convs']
    src_e = edge_index[0, 0::2]
    dst_e = edge_index[1, 0::2]
    src_o = edge_index[0, 1::2]
    dst_o = edge_index[1, 1::2]
    ea_e = edge_attr[0::2]
    ea_o = edge_attr[1::2]

    eprojs = [
        _eproj(ea_e if l % 2 == 0 else ea_o, convs[l]['We'])
        for l in range(NL)
    ]

    h = _fnn(x, params['fnn_in'], HC)
    for l in range(NL):
        p = convs[l]
        s, d = (src_e, dst_e) if l % 2 == 0 else (src_o, dst_o)
        q4, kv4 = _qkv(h, p)
        q4 = q4.reshape(HEADS * N, HID)
        kv4 = kv4.reshape(HEADS * N, 2 * HID)
        acc0 = _sc_attn(q4, kv4, eprojs[l], s, d, 0)
        acc1 = _sc_attn(q4, kv4, eprojs[l], s, d, 1)
        h = _post(acc0, acc1, h, p)
    return _fnn(h, params['fnn_out'], DOUT)
